# speedup vs baseline: 1.3287x; 1.3287x over previous
import functools

import jax
import jax.numpy as jnp
from jax.experimental import pallas as pl

_EPS = 1e-5


def _router_block(x_ref, emb_ref, idx_ref, prob_ref, *, temp):
    x = x_ref[...]
    m = jnp.mean(x, axis=1, keepdims=True)
    c = x - m
    v = jnp.mean(c * c, axis=1, keepdims=True)
    xn = c * jax.lax.rsqrt(v + _EPS)

    emb = emb_ref[...]
    sim = jax.lax.dot_general(
        xn, emb, (((1,), (1,)), ((), ())), preferred_element_type=jnp.float32
    )

    e = sim.shape[1]
    iota = jax.lax.broadcasted_iota(jnp.int32, sim.shape, 1)

    max1 = jnp.max(sim, axis=1, keepdims=True)
    idx1 = jnp.min(jnp.where(sim == max1, iota, e), axis=1, keepdims=True)
    masked = jnp.where(iota == idx1, -jnp.inf, sim)
    max2 = jnp.max(masked, axis=1, keepdims=True)
    idx2 = jnp.min(jnp.where(masked == max2, iota, e), axis=1, keepdims=True)

    s = jnp.exp((max2 - max1) / temp)
    denom = 1.0 + s
    p1 = 1.0 / denom
    p2 = s / denom

    idx_ref[...] = jnp.concatenate([idx1, idx2], axis=1)
    prob_ref[...] = jnp.concatenate([p1, p2], axis=1)


def kernel(input, expert_embeddings):
    b, s, d = input.shape
    e = expert_embeddings.shape[0]
    n = b * s
    x = input.reshape(n, d)

    block_t = 1024
    grid = (n // block_t,)
    temp = float(d) ** 0.5

    idx, prob = pl.pallas_call(
        functools.partial(_router_block, temp=temp),
        grid=grid,
        in_specs=[
            pl.BlockSpec((block_t, d), lambda i: (i, 0)),
            pl.BlockSpec((e, d), lambda i: (0, 0)),
        ],
        out_specs=[
            pl.BlockSpec((block_t, 2), lambda i: (i, 0)),
            pl.BlockSpec((block_t, 2), lambda i: (i, 0)),
        ],
        out_shape=[
            jax.ShapeDtypeStruct((n, 2), jnp.int32),
            jax.ShapeDtypeStruct((n, 2), jnp.float32),
        ],
    )(x, expert_embeddings)

    return idx.reshape(b, s, 2), prob.reshape(b, s, 2)


# speedup vs baseline: 2.1026x; 1.5824x over previous
import functools

import jax
import jax.numpy as jnp
from jax.experimental import pallas as pl

_EPS = 1e-5


def _router_block(x_ref, emb_ref, out_ref, *, temp):
    x = x_ref[...]
    emb = emb_ref[...]

    m = jnp.mean(x, axis=1, keepdims=True)
    c = x - m
    v = jnp.mean(c * c, axis=1, keepdims=True)
    xn = c * jax.lax.rsqrt(v + _EPS)

    sim = jax.lax.dot_general(
        emb, xn, (((1,), (1,)), ((), ())), preferred_element_type=jnp.float32
    )

    iota = jax.lax.broadcasted_iota(jnp.int32, sim.shape, 0)
    max1 = jnp.max(sim, axis=0, keepdims=True)
    idx1 = jnp.min(jnp.where(sim == max1, iota, 8), axis=0, keepdims=True)
    masked = jnp.where(iota == idx1, -jnp.inf, sim)
    max2 = jnp.max(masked, axis=0, keepdims=True)
    idx2 = jnp.min(jnp.where(masked == max2, iota, 8), axis=0, keepdims=True)

    e2 = jnp.exp((max2 - max1) / temp)
    denom = 1.0 + e2
    p1 = 1.0 / denom
    p2 = e2 / denom

    i1f = idx1.astype(jnp.float32)
    i2f = idx2.astype(jnp.float32)
    out_ref[...] = jnp.concatenate([i1f, i2f, p1, p2, i1f, i2f, p1, p2], axis=0)


def kernel(input, expert_embeddings):
    b, s, d = input.shape
    e = expert_embeddings.shape[0]
    n = b * s
    x = input.reshape(n, d)

    block_t = 1024
    grid = (n // block_t,)
    temp = float(d) ** 0.5

    out = pl.pallas_call(
        functools.partial(_router_block, temp=temp),
        grid=grid,
        in_specs=[
            pl.BlockSpec((block_t, d), lambda i: (i, 0)),
            pl.BlockSpec((e, d), lambda i: (0, 0)),
        ],
        out_specs=pl.BlockSpec((8, block_t), lambda i: (0, i)),
        out_shape=jax.ShapeDtypeStruct((8, n), jnp.float32),
    )(x, expert_embeddings)

    idx = out[0:2, :].astype(jnp.int32).T.reshape(b, s, 2)
    probs = out[2:4, :].T.reshape(b, s, 2)
    return idx, probs
